# Initial kernel scaffold; baseline (speedup 1.0000x reference)
import jax
import jax.numpy as jnp
from jax.experimental import pallas as pl


def kernel(H_t, real_edges_t, negative_edges_t):
    raise NotImplementedError("write your pallas kernel here")


# speedup vs baseline: 1.5317x; 1.5317x over previous
import functools

import jax
import jax.numpy as jnp
from jax import lax
from jax.experimental import pallas as pl
from jax.experimental.pallas import tpu as pltpu
from jax.experimental.pallas import tpu_sc as plsc

N_NODES = 10000
D = 128
E = 320000
LANES = 16
CHUNK = 128
GROUPS = CHUNK // LANES
NW = 32
E_PAD = 327680
N_ROWS = E_PAD // CHUNK
RPW = N_ROWS // NW


def _rsqrt_nr(p):
    i = lax.bitcast_convert_type(p, jnp.int32)
    i = jnp.int32(0x5F3759DF) - jnp.right_shift(i, jnp.full((LANES,), 1, jnp.int32))
    y = lax.bitcast_convert_type(i, jnp.float32)
    for _ in range(4):
        y = y * (jnp.float32(1.5) - jnp.float32(0.5) * p * y * y)
    return y


def _make_sc_kernel():
    mesh = plsc.VectorSubcoreMesh(core_axis_name="c", subcore_axis_name="s")

    @functools.partial(
        pl.kernel,
        mesh=mesh,
        compiler_params=pltpu.CompilerParams(needs_layout_passes=False),
        out_type=[
            jax.ShapeDtypeStruct((N_ROWS, CHUNK), jnp.float32),
            jax.ShapeDtypeStruct((N_ROWS, CHUNK), jnp.float32),
        ],
        scratch_types=[
            pltpu.VMEM((RPW, CHUNK), jnp.int32),
            pltpu.VMEM((RPW, CHUNK), jnp.int32),
            pltpu.VMEM((CHUNK, D), jnp.float32),
            pltpu.VMEM((CHUNK, D), jnp.float32),
            pltpu.VMEM((RPW, CHUNK), jnp.float32),
            pltpu.SemaphoreType.DMA,
            pltpu.SemaphoreType.DMA,
        ],
    )
    def link_pred(h_hbm, rs_hbm, rd_hbm, ns_hbm, nd_hbm,
                  out_r_hbm, out_n_hbm,
                  sidx, didx, srow, drow, outv, sem_s, sem_d):
        wid = lax.axis_index("s") * 2 + lax.axis_index("c")
        base = wid * RPW

        for s_hbm, d_hbm, o_hbm in ((rs_hbm, rd_hbm, out_r_hbm),
                                    (ns_hbm, nd_hbm, out_n_hbm)):
            pltpu.sync_copy(s_hbm.at[pl.ds(base, RPW)], sidx)
            pltpu.sync_copy(d_hbm.at[pl.ds(base, RPW)], didx)

            def chunk_body(k, _):
                cs = pltpu.async_copy(h_hbm.at[sidx.at[k]], srow, sem_s)
                cd = pltpu.async_copy(h_hbm.at[didx.at[k]], drow, sem_d)
                cs.wait()
                cd.wait()

                lane = lax.iota(jnp.int32, LANES)

                def group_body(g, _):
                    dotv = jnp.zeros((LANES,), jnp.float32)
                    av = jnp.zeros((LANES,), jnp.float32)
                    bv = jnp.zeros((LANES,), jnp.float32)
                    for u in range(LANES):
                        e = g * jnp.int32(LANES) + jnp.int32(u)
                        num = jnp.zeros((LANES,), jnp.float32)
                        na = jnp.zeros((LANES,), jnp.float32)
                        nb = jnp.zeros((LANES,), jnp.float32)
                        for b in range(D // LANES):
                            s = srow[e, pl.ds(b * LANES, LANES)]
                            t = drow[e, pl.ds(b * LANES, LANES)]
                            num = num + s * t
                            na = na + s * s
                            nb = nb + t * t
                        m = lane == jnp.int32(u)
                        dotv = jnp.where(m, jnp.full((LANES,), jnp.sum(num)), dotv)
                        av = jnp.where(m, jnp.full((LANES,), jnp.sum(na)), av)
                        bv = jnp.where(m, jnp.full((LANES,), jnp.sum(nb)), bv)
                    p = av * bv
                    eps2 = jnp.float32(1e-12)
                    sim = jnp.where(p < eps2,
                                    dotv * jnp.float32(1e6),
                                    dotv * _rsqrt_nr(p))
                    outv[k, pl.ds(g * LANES, LANES)] = sim
                    return 0

                lax.fori_loop(0, GROUPS, group_body, 0)
                return 0

            lax.fori_loop(0, RPW, chunk_body, 0)
            pltpu.sync_copy(outv, o_hbm.at[pl.ds(base, RPW)])

    return link_pred


_sc_kernel = _make_sc_kernel()


def kernel(H_t, real_edges_t, negative_edges_t):
    pad = E_PAD - E
    def prep(v):
        return jnp.pad(v, (0, pad)).reshape(N_ROWS, CHUNK)
    rs = prep(real_edges_t[0])
    rd = prep(real_edges_t[1])
    ns = prep(negative_edges_t[0])
    nd = prep(negative_edges_t[1])
    out_r, out_n = _sc_kernel(H_t, rs, rd, ns, nd)
    return out_r.reshape(-1)[:E], out_n.reshape(-1)[:E]
